# baseline (speedup 1.0000x reference)
import jax
import jax.numpy as jnp
from jax import lax
from jax.experimental import pallas as pl
from jax.experimental.pallas import tpu as pltpu

B = 16
D = 64
K_DB = 1_000_000
K_TOP = 10
S = 8192
G = (K_DB + S - 1) // S
NCOL = 128
NGRP = S // NCOL
NC = G * NCOL
FEAT_IN = 150528
FEAT_CHUNK = 7168
FEAT_G = FEAT_IN // FEAT_CHUNK
T = FEAT_G + G
IMAX = jnp.iinfo(jnp.int32).max


def _tree_reduce(fn, xs):
    while len(xs) > 1:
        nxt = [fn(xs[i], xs[i + 1]) for i in range(0, len(xs) - 1, 2)]
        if len(xs) % 2:
            nxt.append(xs[-1])
        xs = nxt
    return xs[0]


def _extract_topk(v, i, n):
    outv, outi = [], []
    for _ in range(n):
        m = jnp.max(v, axis=1, keepdims=True)
        am = jnp.min(jnp.where(v == m, i, IMAX), axis=1, keepdims=True)
        outv.append(m)
        outi.append(am)
        v = jnp.where((v == m) & (i == am), -jnp.inf, v)
    return jnp.concatenate(outv, axis=1), jnp.concatenate(outi, axis=1)


def _score_block(feat, db_ref, g):
    s = lax.dot_general(feat, db_ref[...], (((1,), (1,)), ((), ())),
                        preferred_element_type=jnp.float32)
    gidx = lax.broadcasted_iota(jnp.int32, (B, S), 1) + g * S
    return jnp.where(gidx < K_DB, s, -jnp.inf), gidx


def _main_kernel(x_ref, w_ref, db_ref, cv_ref, ci_ref, c2_ref, fo_ref,
                 feat_ref):
    t = pl.program_id(0)

    @pl.when(t == 0)
    def _():
        feat_ref[...] = jnp.zeros_like(feat_ref)

    @pl.when(t < FEAT_G)
    def _():
        feat_ref[...] += jnp.dot(x_ref[...], w_ref[...],
                                 preferred_element_type=jnp.float32)

    @pl.when(t == FEAT_G - 1)
    def _():
        fo_ref[...] = feat_ref[...]

    @pl.when(t >= FEAT_G)
    def _():
        g = t - FEAT_G
        s, _ = _score_block(feat_ref[...], db_ref, g)

        parts = [s[:, j * NCOL:(j + 1) * NCOL] for j in range(NGRP)]
        colmax = _tree_reduce(jnp.maximum, parts)
        colj = _tree_reduce(jnp.minimum,
                            [jnp.where(parts[j] == colmax, j, NGRP)
                             for j in range(NGRP)])
        col2 = _tree_reduce(jnp.maximum,
                            [jnp.where((parts[j] == colmax) & (colj == j),
                                       -jnp.inf, parts[j])
                             for j in range(NGRP)])
        lane = lax.broadcasted_iota(jnp.int32, (B, NCOL), 1)
        cv_ref[...] = colmax
        ci_ref[...] = g * S + colj * NCOL + lane
        c2_ref[...] = col2


def _merge_kernel(cv_ref, ci_ref, c2_ref, vals_ref, idx_ref, bad_ref):
    fv, fi = _extract_topk(cv_ref[...], ci_ref[...], K_TOP)
    vals_ref[...] = fv
    idx_ref[...] = fi
    t10 = fv[:, K_TOP - 1:K_TOP]
    bad = jnp.any(c2_ref[...] >= t10)
    bad_ref[...] = jnp.full((1, 1), bad, jnp.int32)


def _exact_kernel(feat_ref, db_ref, vals_ref, idx_ref, cv_ref, ci_ref):
    g = pl.program_id(0)
    s, gidx = _score_block(feat_ref[...], db_ref, g)
    cv, ci = _extract_topk(s, gidx, K_TOP)
    pad_v = jnp.full((B, NCOL - K_TOP), -jnp.inf, jnp.float32)
    pad_i = jnp.full((B, NCOL - K_TOP), IMAX, jnp.int32)
    cv_ref[:, pl.ds(g * NCOL, NCOL)] = jnp.concatenate([cv, pad_v], 1)
    ci_ref[:, pl.ds(g * NCOL, NCOL)] = jnp.concatenate([ci, pad_i], 1)

    @pl.when(g == G - 1)
    def _():
        fv, fi = _extract_topk(cv_ref[...], ci_ref[...], K_TOP)
        vals_ref[...] = fv
        idx_ref[...] = fi


def kernel(image, k, W, database):
    x = image.reshape(B, FEAT_IN)
    x = lax.concatenate([x[:, :FEAT_CHUNK], x[:, FEAT_CHUNK:]], 1)

    def _x_map(t):
        return (0, jnp.minimum(t, FEAT_G - 1))

    def _w_map(t):
        return (jnp.minimum(t, FEAT_G - 1), 0)

    def _db_map(t):
        return (jnp.maximum(t - FEAT_G, 0), 0)

    def _out_map(t):
        return (0, jnp.maximum(t - FEAT_G, 0))

    cv, ci, c2, feat = pl.pallas_call(
        _main_kernel,
        grid=(T,),
        in_specs=[
            pl.BlockSpec((B, FEAT_CHUNK), _x_map),
            pl.BlockSpec((FEAT_CHUNK, D), _w_map),
            pl.BlockSpec((S, D), _db_map),
        ],
        out_specs=[
            pl.BlockSpec((B, NCOL), _out_map),
            pl.BlockSpec((B, NCOL), _out_map),
            pl.BlockSpec((B, NCOL), _out_map),
            pl.BlockSpec((B, D), lambda t: (0, 0)),
        ],
        out_shape=[
            jax.ShapeDtypeStruct((B, NC), jnp.float32),
            jax.ShapeDtypeStruct((B, NC), jnp.int32),
            jax.ShapeDtypeStruct((B, NC), jnp.float32),
            jax.ShapeDtypeStruct((B, D), jnp.float32),
        ],
        scratch_shapes=[
            pltpu.VMEM((B, D), jnp.float32),
        ],
        compiler_params=pltpu.CompilerParams(
            dimension_semantics=("arbitrary",)),
    )(x, W, database)

    vals, idx, bad = pl.pallas_call(
        _merge_kernel,
        in_specs=[
            pl.BlockSpec((B, NC), lambda: (0, 0)),
            pl.BlockSpec((B, NC), lambda: (0, 0)),
            pl.BlockSpec((B, NC), lambda: (0, 0)),
        ],
        out_specs=[
            pl.BlockSpec((B, K_TOP), lambda: (0, 0)),
            pl.BlockSpec((B, K_TOP), lambda: (0, 0)),
            pl.BlockSpec((1, 1), lambda: (0, 0)),
        ],
        out_shape=[
            jax.ShapeDtypeStruct((B, K_TOP), jnp.float32),
            jax.ShapeDtypeStruct((B, K_TOP), jnp.int32),
            jax.ShapeDtypeStruct((1, 1), jnp.int32),
        ],
    )(cv, ci, c2)

    def _slow_path():
        return pl.pallas_call(
            _exact_kernel,
            grid=(G,),
            in_specs=[
                pl.BlockSpec((B, D), lambda g: (0, 0)),
                pl.BlockSpec((S, D), lambda g: (g, 0)),
            ],
            out_specs=[
                pl.BlockSpec((B, K_TOP), lambda g: (0, 0)),
                pl.BlockSpec((B, K_TOP), lambda g: (0, 0)),
            ],
            out_shape=[
                jax.ShapeDtypeStruct((B, K_TOP), jnp.float32),
                jax.ShapeDtypeStruct((B, K_TOP), jnp.int32),
            ],
            scratch_shapes=[
                pltpu.VMEM((B, NC), jnp.float32),
                pltpu.VMEM((B, NC), jnp.int32),
            ],
            compiler_params=pltpu.CompilerParams(
                dimension_semantics=("arbitrary",)),
        )(feat, database)

    return lax.cond(bad[0, 0] != 0, _slow_path, lambda: (vals, idx))


# speedup vs baseline: 1.0563x; 1.0563x over previous
import jax
import jax.numpy as jnp
from jax import lax
from jax.experimental import pallas as pl
from jax.experimental.pallas import tpu as pltpu

B = 16
D = 64
K_DB = 1_000_000
K_TOP = 10
S = 16384
G = (K_DB + S - 1) // S
NCOL = 128
NGRP = S // NCOL
NC = G * NCOL
FEAT_IN = 150528
FEAT_CHUNK = 7168
FEAT_G = FEAT_IN // FEAT_CHUNK
T = FEAT_G + G
IMAX = jnp.iinfo(jnp.int32).max


def _tree_reduce(fn, xs):
    while len(xs) > 1:
        nxt = [fn(xs[i], xs[i + 1]) for i in range(0, len(xs) - 1, 2)]
        if len(xs) % 2:
            nxt.append(xs[-1])
        xs = nxt
    return xs[0]


def _extract_topk(v, i, n):
    outv, outi = [], []
    for _ in range(n):
        m = jnp.max(v, axis=1, keepdims=True)
        am = jnp.min(jnp.where(v == m, i, IMAX), axis=1, keepdims=True)
        outv.append(m)
        outi.append(am)
        v = jnp.where((v == m) & (i == am), -jnp.inf, v)
    return jnp.concatenate(outv, axis=1), jnp.concatenate(outi, axis=1)


def _score_block(feat, db_ref, g):
    s = lax.dot_general(feat, db_ref[...], (((1,), (1,)), ((), ())),
                        preferred_element_type=jnp.float32)
    gidx = lax.broadcasted_iota(jnp.int32, (B, S), 1) + g * S
    return jnp.where(gidx < K_DB, s, -jnp.inf), gidx


def _main_kernel(x_ref, w_ref, db_ref, cv_ref, ci_ref, c2_ref, fo_ref,
                 feat_ref):
    t = pl.program_id(0)

    @pl.when(t == 0)
    def _():
        feat_ref[...] = jnp.zeros_like(feat_ref)

    @pl.when(t < FEAT_G)
    def _():
        feat_ref[...] += jnp.dot(x_ref[...], w_ref[...],
                                 preferred_element_type=jnp.float32)

    @pl.when(t == FEAT_G - 1)
    def _():
        fo_ref[...] = feat_ref[...]

    @pl.when(t >= FEAT_G)
    def _():
        g = t - FEAT_G
        s, _ = _score_block(feat_ref[...], db_ref, g)

        nodes = [(s[:, j * NCOL:(j + 1) * NCOL], None, None)
                 for j in range(NGRP)]
        level = 0
        while len(nodes) > 1:
            nxt = []
            for a in range(0, len(nodes) - 1, 2):
                (m1, j1, s1), (m2, j2, s2) = nodes[a], nodes[a + 1]
                ge = m1 >= m2
                m = jnp.maximum(m1, m2)
                if level == 0:
                    j = jnp.where(ge, a, a + 1)
                    s2nd = jnp.minimum(m1, m2)
                else:
                    j = jnp.where(ge, j1, j2)
                    s2nd = jnp.maximum(jnp.minimum(m1, m2),
                                       jnp.maximum(s1, s2))
                nxt.append((m, j, s2nd))
            if len(nodes) % 2:
                nodes.append(None)
            nodes = nxt
            level += 1
        colmax, colj, col2 = nodes[0]
        lane = lax.broadcasted_iota(jnp.int32, (B, NCOL), 1)
        cv_ref[...] = colmax
        ci_ref[...] = g * S + colj * NCOL + lane
        c2_ref[...] = col2


def _merge_kernel(cv_ref, ci_ref, c2_ref, vals_ref, idx_ref, bad_ref):
    fv, fi = _extract_topk(cv_ref[...], ci_ref[...], K_TOP)
    vals_ref[...] = fv
    idx_ref[...] = fi
    t10 = fv[:, K_TOP - 1:K_TOP]
    bad = jnp.any(c2_ref[...] >= t10)
    bad_ref[...] = jnp.full((1, 1), bad, jnp.int32)


def _exact_kernel(feat_ref, db_ref, vals_ref, idx_ref, cv_ref, ci_ref):
    g = pl.program_id(0)
    s, gidx = _score_block(feat_ref[...], db_ref, g)
    cv, ci = _extract_topk(s, gidx, K_TOP)
    pad_v = jnp.full((B, NCOL - K_TOP), -jnp.inf, jnp.float32)
    pad_i = jnp.full((B, NCOL - K_TOP), IMAX, jnp.int32)
    cv_ref[:, pl.ds(g * NCOL, NCOL)] = jnp.concatenate([cv, pad_v], 1)
    ci_ref[:, pl.ds(g * NCOL, NCOL)] = jnp.concatenate([ci, pad_i], 1)

    @pl.when(g == G - 1)
    def _():
        fv, fi = _extract_topk(cv_ref[...], ci_ref[...], K_TOP)
        vals_ref[...] = fv
        idx_ref[...] = fi


def kernel(image, k, W, database):
    x = image.reshape(B, FEAT_IN)

    def _x_map(t):
        return (0, jnp.minimum(t, FEAT_G - 1))

    def _w_map(t):
        return (jnp.minimum(t, FEAT_G - 1), 0)

    def _db_map(t):
        return (jnp.maximum(t - FEAT_G, 0), 0)

    def _out_map(t):
        return (0, jnp.maximum(t - FEAT_G, 0))

    cv, ci, c2, feat = pl.pallas_call(
        _main_kernel,
        grid=(T,),
        in_specs=[
            pl.BlockSpec((B, FEAT_CHUNK), _x_map),
            pl.BlockSpec((FEAT_CHUNK, D), _w_map),
            pl.BlockSpec((S, D), _db_map),
        ],
        out_specs=[
            pl.BlockSpec((B, NCOL), _out_map),
            pl.BlockSpec((B, NCOL), _out_map),
            pl.BlockSpec((B, NCOL), _out_map),
            pl.BlockSpec((B, D), lambda t: (0, 0)),
        ],
        out_shape=[
            jax.ShapeDtypeStruct((B, NC), jnp.float32),
            jax.ShapeDtypeStruct((B, NC), jnp.int32),
            jax.ShapeDtypeStruct((B, NC), jnp.float32),
            jax.ShapeDtypeStruct((B, D), jnp.float32),
        ],
        scratch_shapes=[
            pltpu.VMEM((B, D), jnp.float32),
        ],
        compiler_params=pltpu.CompilerParams(
            dimension_semantics=("arbitrary",)),
    )(x, W, database)

    vals, idx, bad = pl.pallas_call(
        _merge_kernel,
        in_specs=[
            pl.BlockSpec((B, NC), lambda: (0, 0)),
            pl.BlockSpec((B, NC), lambda: (0, 0)),
            pl.BlockSpec((B, NC), lambda: (0, 0)),
        ],
        out_specs=[
            pl.BlockSpec((B, K_TOP), lambda: (0, 0)),
            pl.BlockSpec((B, K_TOP), lambda: (0, 0)),
            pl.BlockSpec((1, 1), lambda: (0, 0)),
        ],
        out_shape=[
            jax.ShapeDtypeStruct((B, K_TOP), jnp.float32),
            jax.ShapeDtypeStruct((B, K_TOP), jnp.int32),
            jax.ShapeDtypeStruct((1, 1), jnp.int32),
        ],
    )(cv, ci, c2)

    def _slow_path():
        return pl.pallas_call(
            _exact_kernel,
            grid=(G,),
            in_specs=[
                pl.BlockSpec((B, D), lambda g: (0, 0)),
                pl.BlockSpec((S, D), lambda g: (g, 0)),
            ],
            out_specs=[
                pl.BlockSpec((B, K_TOP), lambda g: (0, 0)),
                pl.BlockSpec((B, K_TOP), lambda g: (0, 0)),
            ],
            out_shape=[
                jax.ShapeDtypeStruct((B, K_TOP), jnp.float32),
                jax.ShapeDtypeStruct((B, K_TOP), jnp.int32),
            ],
            scratch_shapes=[
                pltpu.VMEM((B, NC), jnp.float32),
                pltpu.VMEM((B, NC), jnp.int32),
            ],
            compiler_params=pltpu.CompilerParams(
                dimension_semantics=("arbitrary",)),
        )(feat, database)

    return lax.cond(bad[0, 0] != 0, _slow_path, lambda: (vals, idx))


# speedup vs baseline: 1.0564x; 1.0002x over previous
import jax
import jax.numpy as jnp
from jax import lax
from jax.experimental import pallas as pl
from jax.experimental.pallas import tpu as pltpu

B = 16
D = 64
K_DB = 1_000_000
K_TOP = 10
S = 16384
G = (K_DB + S - 1) // S
NCOL = 128
NGRP = S // NCOL
NC = G * NCOL
FEAT_IN = 150528
FEAT_CHUNK = 7168
FEAT_G = FEAT_IN // FEAT_CHUNK
T = FEAT_G + G
IMAX = jnp.iinfo(jnp.int32).max


def _tree_reduce(fn, xs):
    while len(xs) > 1:
        nxt = [fn(xs[i], xs[i + 1]) for i in range(0, len(xs) - 1, 2)]
        if len(xs) % 2:
            nxt.append(xs[-1])
        xs = nxt
    return xs[0]


def _extract_topk(v, i, n):
    outv, outi = [], []
    for _ in range(n):
        m = jnp.max(v, axis=1, keepdims=True)
        am = jnp.min(jnp.where(v == m, i, IMAX), axis=1, keepdims=True)
        outv.append(m)
        outi.append(am)
        v = jnp.where((v == m) & (i == am), -jnp.inf, v)
    return jnp.concatenate(outv, axis=1), jnp.concatenate(outi, axis=1)


def _score_block(feat, db_ref, g):
    s = lax.dot_general(feat, db_ref[...], (((1,), (1,)), ((), ())),
                        preferred_element_type=jnp.float32)
    gidx = lax.broadcasted_iota(jnp.int32, (B, S), 1) + g * S
    return jnp.where(gidx < K_DB, s, -jnp.inf), gidx


def _main_kernel(x_ref, w_ref, db_ref, cv_ref, ci_ref, c2_ref, fo_ref,
                 feat_ref):
    t = pl.program_id(0)

    @pl.when(t == 0)
    def _():
        feat_ref[...] = jnp.zeros_like(feat_ref)

    @pl.when(t < FEAT_G)
    def _():
        feat_ref[...] += jnp.dot(x_ref[...], w_ref[...],
                                 preferred_element_type=jnp.float32)

    @pl.when(t == FEAT_G - 1)
    def _():
        fo_ref[...] = feat_ref[...]

    @pl.when(t >= FEAT_G)
    def _():
        g = t - FEAT_G
        s, _ = _score_block(feat_ref[...], db_ref, g)

        nodes = [(s[:, j * NCOL:(j + 1) * NCOL], None, None)
                 for j in range(NGRP)]
        level = 0
        while len(nodes) > 1:
            nxt = []
            for a in range(0, len(nodes) - 1, 2):
                (m1, j1, s1), (m2, j2, s2) = nodes[a], nodes[a + 1]
                ge = m1 >= m2
                m = jnp.maximum(m1, m2)
                if level == 0:
                    j = jnp.where(ge, a, a + 1)
                    s2nd = jnp.minimum(m1, m2)
                else:
                    j = jnp.where(ge, j1, j2)
                    s2nd = jnp.maximum(jnp.minimum(m1, m2),
                                       jnp.maximum(s1, s2))
                nxt.append((m, j, s2nd))
            if len(nodes) % 2:
                nodes.append(None)
            nodes = nxt
            level += 1
        colmax, colj, col2 = nodes[0]
        lane = lax.broadcasted_iota(jnp.int32, (B, NCOL), 1)
        cv_ref[...] = colmax
        ci_ref[...] = g * S + colj * NCOL + lane
        c2_ref[...] = col2


def _merge_kernel(cv_ref, ci_ref, c2_ref, vals_ref, idx_ref, bad_ref):
    fv, fi = _extract_topk(cv_ref[...], ci_ref[...], K_TOP)
    vals_ref[...] = fv
    idx_ref[...] = fi
    t10 = fv[:, K_TOP - 1:K_TOP]
    bad = jnp.any(c2_ref[...] >= t10)
    bad_ref[...] = jnp.full((1, 1), bad, jnp.int32)


def _exact_kernel(feat_ref, db_ref, vals_ref, idx_ref, cv_ref, ci_ref):
    g = pl.program_id(0)
    s, gidx = _score_block(feat_ref[...], db_ref, g)
    cv, ci = _extract_topk(s, gidx, K_TOP)
    pad_v = jnp.full((B, NCOL - K_TOP), -jnp.inf, jnp.float32)
    pad_i = jnp.full((B, NCOL - K_TOP), IMAX, jnp.int32)
    cv_ref[:, pl.ds(g * NCOL, NCOL)] = jnp.concatenate([cv, pad_v], 1)
    ci_ref[:, pl.ds(g * NCOL, NCOL)] = jnp.concatenate([ci, pad_i], 1)

    @pl.when(g == G - 1)
    def _():
        fv, fi = _extract_topk(cv_ref[...], ci_ref[...], K_TOP)
        vals_ref[...] = fv
        idx_ref[...] = fi


def kernel(image, k, W, database):
    xu = lax.bitcast_convert_type(image, jnp.uint32)
    x = lax.bitcast_convert_type(xu.reshape(B, FEAT_IN), jnp.float32)

    def _x_map(t):
        return (0, jnp.minimum(t, FEAT_G - 1))

    def _w_map(t):
        return (jnp.minimum(t, FEAT_G - 1), 0)

    def _db_map(t):
        return (jnp.maximum(t - FEAT_G, 0), 0)

    def _out_map(t):
        return (0, jnp.maximum(t - FEAT_G, 0))

    cv, ci, c2, feat = pl.pallas_call(
        _main_kernel,
        grid=(T,),
        in_specs=[
            pl.BlockSpec((B, FEAT_CHUNK), _x_map),
            pl.BlockSpec((FEAT_CHUNK, D), _w_map),
            pl.BlockSpec((S, D), _db_map),
        ],
        out_specs=[
            pl.BlockSpec((B, NCOL), _out_map),
            pl.BlockSpec((B, NCOL), _out_map),
            pl.BlockSpec((B, NCOL), _out_map),
            pl.BlockSpec((B, D), lambda t: (0, 0)),
        ],
        out_shape=[
            jax.ShapeDtypeStruct((B, NC), jnp.float32),
            jax.ShapeDtypeStruct((B, NC), jnp.int32),
            jax.ShapeDtypeStruct((B, NC), jnp.float32),
            jax.ShapeDtypeStruct((B, D), jnp.float32),
        ],
        scratch_shapes=[
            pltpu.VMEM((B, D), jnp.float32),
        ],
        compiler_params=pltpu.CompilerParams(
            dimension_semantics=("arbitrary",)),
    )(x, W, database)

    vals, idx, bad = pl.pallas_call(
        _merge_kernel,
        in_specs=[
            pl.BlockSpec((B, NC), lambda: (0, 0)),
            pl.BlockSpec((B, NC), lambda: (0, 0)),
            pl.BlockSpec((B, NC), lambda: (0, 0)),
        ],
        out_specs=[
            pl.BlockSpec((B, K_TOP), lambda: (0, 0)),
            pl.BlockSpec((B, K_TOP), lambda: (0, 0)),
            pl.BlockSpec((1, 1), lambda: (0, 0)),
        ],
        out_shape=[
            jax.ShapeDtypeStruct((B, K_TOP), jnp.float32),
            jax.ShapeDtypeStruct((B, K_TOP), jnp.int32),
            jax.ShapeDtypeStruct((1, 1), jnp.int32),
        ],
    )(cv, ci, c2)

    def _slow_path():
        return pl.pallas_call(
            _exact_kernel,
            grid=(G,),
            in_specs=[
                pl.BlockSpec((B, D), lambda g: (0, 0)),
                pl.BlockSpec((S, D), lambda g: (g, 0)),
            ],
            out_specs=[
                pl.BlockSpec((B, K_TOP), lambda g: (0, 0)),
                pl.BlockSpec((B, K_TOP), lambda g: (0, 0)),
            ],
            out_shape=[
                jax.ShapeDtypeStruct((B, K_TOP), jnp.float32),
                jax.ShapeDtypeStruct((B, K_TOP), jnp.int32),
            ],
            scratch_shapes=[
                pltpu.VMEM((B, NC), jnp.float32),
                pltpu.VMEM((B, NC), jnp.int32),
            ],
            compiler_params=pltpu.CompilerParams(
                dimension_semantics=("arbitrary",)),
        )(feat, database)

    return lax.cond(bad[0, 0] != 0, _slow_path, lambda: (vals, idx))


# speedup vs baseline: 1.0986x; 1.0399x over previous
import jax
import jax.numpy as jnp
from jax import lax
from jax.experimental import pallas as pl
from jax.experimental.pallas import tpu as pltpu

B = 16
D = 64
K_DB = 1_000_000
K_TOP = 10
S = 16384
G = (K_DB + S - 1) // S
NCOL = 128
NGRP = S // NCOL
NC = G * NCOL
FEAT_IN = 150528
FEAT_ROWS = 1176
FEAT_RCH = 56
FEAT_G = FEAT_ROWS // FEAT_RCH
T = FEAT_G + G
IMAX = jnp.iinfo(jnp.int32).max


def _tree_reduce(fn, xs):
    while len(xs) > 1:
        nxt = [fn(xs[i], xs[i + 1]) for i in range(0, len(xs) - 1, 2)]
        if len(xs) % 2:
            nxt.append(xs[-1])
        xs = nxt
    return xs[0]


def _extract_topk(v, i, n):
    outv, outi = [], []
    for _ in range(n):
        m = jnp.max(v, axis=1, keepdims=True)
        am = jnp.min(jnp.where(v == m, i, IMAX), axis=1, keepdims=True)
        outv.append(m)
        outi.append(am)
        v = jnp.where((v == m) & (i == am), -jnp.inf, v)
    return jnp.concatenate(outv, axis=1), jnp.concatenate(outi, axis=1)


def _score_block(feat, db_ref, g):
    s = lax.dot_general(feat, db_ref[...], (((1,), (1,)), ((), ())),
                        preferred_element_type=jnp.float32)
    gidx = lax.broadcasted_iota(jnp.int32, (B, S), 1) + g * S
    return jnp.where(gidx < K_DB, s, -jnp.inf), gidx


def _main_kernel(x_ref, w_ref, db_ref, cv_ref, ci_ref, c2_ref, fo_ref,
                 feat_ref):
    t = pl.program_id(0)

    @pl.when(t == 0)
    def _():
        feat_ref[...] = jnp.zeros_like(feat_ref)

    @pl.when(t < FEAT_G)
    def _():
        acc = feat_ref[...]
        for q in range(FEAT_RCH):
            acc += jnp.dot(x_ref[:, q, :], w_ref[q],
                           preferred_element_type=jnp.float32)
        feat_ref[...] = acc

    @pl.when(t == FEAT_G - 1)
    def _():
        fo_ref[...] = feat_ref[...]

    @pl.when(t >= FEAT_G)
    def _():
        g = t - FEAT_G
        s, _ = _score_block(feat_ref[...], db_ref, g)

        nodes = [(s[:, j * NCOL:(j + 1) * NCOL], None, None)
                 for j in range(NGRP)]
        level = 0
        while len(nodes) > 1:
            nxt = []
            for a in range(0, len(nodes) - 1, 2):
                (m1, j1, s1), (m2, j2, s2) = nodes[a], nodes[a + 1]
                ge = m1 >= m2
                m = jnp.maximum(m1, m2)
                if level == 0:
                    j = jnp.where(ge, a, a + 1)
                    s2nd = jnp.minimum(m1, m2)
                else:
                    j = jnp.where(ge, j1, j2)
                    s2nd = jnp.maximum(jnp.minimum(m1, m2),
                                       jnp.maximum(s1, s2))
                nxt.append((m, j, s2nd))
            if len(nodes) % 2:
                nodes.append(None)
            nodes = nxt
            level += 1
        colmax, colj, col2 = nodes[0]
        lane = lax.broadcasted_iota(jnp.int32, (B, NCOL), 1)
        cv_ref[...] = colmax
        ci_ref[...] = g * S + colj * NCOL + lane
        c2_ref[...] = col2


def _merge_kernel(cv_ref, ci_ref, c2_ref, vals_ref, idx_ref, bad_ref):
    fv, fi = _extract_topk(cv_ref[...], ci_ref[...], K_TOP)
    vals_ref[...] = fv
    idx_ref[...] = fi
    t10 = fv[:, K_TOP - 1:K_TOP]
    bad = jnp.any(c2_ref[...] >= t10)
    bad_ref[...] = jnp.full((1, 1), bad, jnp.int32)


def _exact_kernel(feat_ref, db_ref, vals_ref, idx_ref, cv_ref, ci_ref):
    g = pl.program_id(0)
    s, gidx = _score_block(feat_ref[...], db_ref, g)
    cv, ci = _extract_topk(s, gidx, K_TOP)
    pad_v = jnp.full((B, NCOL - K_TOP), -jnp.inf, jnp.float32)
    pad_i = jnp.full((B, NCOL - K_TOP), IMAX, jnp.int32)
    cv_ref[:, pl.ds(g * NCOL, NCOL)] = jnp.concatenate([cv, pad_v], 1)
    ci_ref[:, pl.ds(g * NCOL, NCOL)] = jnp.concatenate([ci, pad_i], 1)

    @pl.when(g == G - 1)
    def _():
        fv, fi = _extract_topk(cv_ref[...], ci_ref[...], K_TOP)
        vals_ref[...] = fv
        idx_ref[...] = fi


def kernel(image, k, W, database):
    x = image.reshape(B, FEAT_ROWS, 128)
    w3 = W.reshape(FEAT_ROWS, 128, D)

    def _x_map(t):
        return (0, jnp.minimum(t, FEAT_G - 1), 0)

    def _w_map(t):
        return (jnp.minimum(t, FEAT_G - 1), 0, 0)

    def _db_map(t):
        return (jnp.maximum(t - FEAT_G, 0), 0)

    def _out_map(t):
        return (0, jnp.maximum(t - FEAT_G, 0))

    cv, ci, c2, feat = pl.pallas_call(
        _main_kernel,
        grid=(T,),
        in_specs=[
            pl.BlockSpec((B, FEAT_RCH, 128), _x_map),
            pl.BlockSpec((FEAT_RCH, 128, D), _w_map),
            pl.BlockSpec((S, D), _db_map),
        ],
        out_specs=[
            pl.BlockSpec((B, NCOL), _out_map),
            pl.BlockSpec((B, NCOL), _out_map),
            pl.BlockSpec((B, NCOL), _out_map),
            pl.BlockSpec((B, D), lambda t: (0, 0)),
        ],
        out_shape=[
            jax.ShapeDtypeStruct((B, NC), jnp.float32),
            jax.ShapeDtypeStruct((B, NC), jnp.int32),
            jax.ShapeDtypeStruct((B, NC), jnp.float32),
            jax.ShapeDtypeStruct((B, D), jnp.float32),
        ],
        scratch_shapes=[
            pltpu.VMEM((B, D), jnp.float32),
        ],
        compiler_params=pltpu.CompilerParams(
            dimension_semantics=("arbitrary",)),
    )(x, w3, database)

    vals, idx, bad = pl.pallas_call(
        _merge_kernel,
        in_specs=[
            pl.BlockSpec((B, NC), lambda: (0, 0)),
            pl.BlockSpec((B, NC), lambda: (0, 0)),
            pl.BlockSpec((B, NC), lambda: (0, 0)),
        ],
        out_specs=[
            pl.BlockSpec((B, K_TOP), lambda: (0, 0)),
            pl.BlockSpec((B, K_TOP), lambda: (0, 0)),
            pl.BlockSpec((1, 1), lambda: (0, 0)),
        ],
        out_shape=[
            jax.ShapeDtypeStruct((B, K_TOP), jnp.float32),
            jax.ShapeDtypeStruct((B, K_TOP), jnp.int32),
            jax.ShapeDtypeStruct((1, 1), jnp.int32),
        ],
    )(cv, ci, c2)

    def _slow_path():
        return pl.pallas_call(
            _exact_kernel,
            grid=(G,),
            in_specs=[
                pl.BlockSpec((B, D), lambda g: (0, 0)),
                pl.BlockSpec((S, D), lambda g: (g, 0)),
            ],
            out_specs=[
                pl.BlockSpec((B, K_TOP), lambda g: (0, 0)),
                pl.BlockSpec((B, K_TOP), lambda g: (0, 0)),
            ],
            out_shape=[
                jax.ShapeDtypeStruct((B, K_TOP), jnp.float32),
                jax.ShapeDtypeStruct((B, K_TOP), jnp.int32),
            ],
            scratch_shapes=[
                pltpu.VMEM((B, NC), jnp.float32),
                pltpu.VMEM((B, NC), jnp.int32),
            ],
            compiler_params=pltpu.CompilerParams(
                dimension_semantics=("arbitrary",)),
        )(feat, database)

    return lax.cond(bad[0, 0] != 0, _slow_path, lambda: (vals, idx))
